# baseline (speedup 1.0000x reference)
import jax
import jax.numpy as jnp
from jax import lax
from jax.experimental import pallas as pl
from jax.experimental.pallas import tpu as pltpu
from jax.experimental.pallas import tpu_sc as plsc

_B = 16384
_L = 50
_D = 64
_NC = 2
_NS = 16
_NW = _NC * _NS
_U = 16
_NI = _U * _L
_UPW = _B // _NW
_NCH = _UPW // _U
_PST = 17
_PRS = 64 * _PST + 16
_LP = 56


def _sc_body(users_hbm, items_hbm, ut_hbm, it_hbm, out_hbm,
             uidx_v, iidx_v, urows_v, irows_v, prow_v, scores_v,
             isem0, isem1, rsem0, rsem1, osem0, osem1):
    wid = lax.axis_index("s") * _NC + lax.axis_index("c")
    base_row = wid * _UPW
    isems = (isem0, isem1)
    rsems = (rsem0, rsem1)
    osems = (osem0, osem1)

    def issue_idx(ch, buf):
        row0 = base_row + ch * _U
        pltpu.async_copy(users_hbm.at[pl.ds(row0, _U)], uidx_v.at[buf],
                         isems[buf])
        pltpu.async_copy(items_hbm.at[pl.ds(row0, _U)], iidx_v.at[buf],
                         isems[buf])

    def drain_idx(buf):
        pltpu.make_async_copy(users_hbm.at[pl.ds(0, _U)], uidx_v.at[buf],
                              isems[buf]).wait()
        pltpu.make_async_copy(items_hbm.at[pl.ds(0, _U)], iidx_v.at[buf],
                              isems[buf]).wait()

    def issue_rows(buf):
        pltpu.async_copy(ut_hbm.at[uidx_v.at[buf]], urows_v.at[buf],
                         rsems[buf])
        for c in range(_U):
            pltpu.async_copy(
                it_hbm.at[iidx_v.at[buf, c, pl.ds(0, _LP)]],
                irows_v.at[buf, pl.ds(c * _LP, _LP)], rsems[buf])

    def drain_rows(buf):
        pltpu.make_async_copy(ut_hbm.at[pl.ds(0, _U)], urows_v.at[buf],
                              rsems[buf]).wait()
        pltpu.make_async_copy(it_hbm.at[pl.ds(0, _U * _LP)], irows_v.at[buf],
                              rsems[buf]).wait()

    def drain_out(buf):
        pltpu.make_async_copy(scores_v.at[buf], out_hbm.at[pl.ds(0, _U)],
                              osems[buf]).wait()

    lane17 = lax.iota(jnp.int32, 16) * _PST

    def compute(ch, buf):
        def user_body(c, carry2):
            u0 = urows_v[buf, c, pl.ds(0, 16)]
            u1 = urows_v[buf, c, pl.ds(16, 16)]
            u2 = urows_v[buf, c, pl.ds(32, 16)]
            u3 = urows_v[buf, c, pl.ds(48, 16)]

            @plsc.parallel_loop(0, _L, unroll=5)
            def _(l):
                n = c * _LP + l
                p = ((irows_v[buf, n, pl.ds(0, 16)] * u0
                      + irows_v[buf, n, pl.ds(16, 16)] * u1)
                     + (irows_v[buf, n, pl.ds(32, 16)] * u2
                        + irows_v[buf, n, pl.ds(48, 16)] * u3))
                prow_v[buf, pl.ds(l * _PST, 16)] = p

            @plsc.parallel_loop(0, 4)
            def _(g):
                col0 = g * (16 * _PST)
                acc = plsc.load_gather(prow_v.at[buf], [lane17 + col0])
                for d in range(1, 16):
                    acc = acc + plsc.load_gather(prow_v.at[buf],
                                                 [lane17 + (col0 + d)])
                scores_v[buf, c, pl.ds(g * 16, 16)] = acc
            return carry2

        lax.fori_loop(0, _U, user_body, 0)
        pltpu.async_copy(scores_v.at[buf],
                         out_hbm.at[pl.ds(base_row + ch * _U, _U)],
                         osems[buf])

    def step(ch, buf, first):
        nbuf = 1 - buf
        drain_idx(nbuf)
        issue_rows(nbuf)
        drain_rows(buf)
        issue_idx(jnp.minimum(ch + 2, _NCH - 1), buf)
        if not first:
            drain_out(buf)
        compute(ch, buf)

    issue_idx(0, 0)
    issue_idx(1, 1)
    drain_idx(0)
    issue_rows(0)

    step(0, 0, True)
    step(1, 1, True)

    def pair_body(i, carry):
        step(2 * i, 0, False)
        step(2 * i + 1, 1, False)
        return carry

    lax.fori_loop(1, _NCH // 2, pair_body, 0)

    drain_idx(1)
    drain_rows(0)
    drain_out(0)
    drain_out(1)


def kernel(users, items, user_table, item_table):
    items64 = jnp.pad(items, ((0, 0), (0, _D - _L)))
    mesh = plsc.VectorSubcoreMesh(core_axis_name="c", subcore_axis_name="s")
    out64 = pl.kernel(
        _sc_body,
        mesh=mesh,
        compiler_params=pltpu.CompilerParams(
            needs_layout_passes=False, use_tc_tiling_on_sc=False),
        out_type=jax.ShapeDtypeStruct((_B, _D), jnp.float32),
        scratch_types=[
            pltpu.VMEM((2, _U), jnp.int32),
            pltpu.VMEM((2, _U, _D), jnp.int32),
            pltpu.VMEM((2, _U, _D), jnp.float32),
            pltpu.VMEM((2, _U * _LP, _D), jnp.float32),
            pltpu.VMEM((2, _PRS), jnp.float32),
            pltpu.VMEM((2, _U, _D), jnp.float32),
            pltpu.SemaphoreType.DMA,
            pltpu.SemaphoreType.DMA,
            pltpu.SemaphoreType.DMA,
            pltpu.SemaphoreType.DMA,
            pltpu.SemaphoreType.DMA,
            pltpu.SemaphoreType.DMA,
        ],
    )(users, items64, user_table, item_table)
    return out64[:, :_L]


# speedup vs baseline: 2.5907x; 2.5907x over previous
import jax
import jax.numpy as jnp
from jax import lax
from jax.experimental import pallas as pl
from jax.experimental.pallas import tpu as pltpu
from jax.experimental.pallas import tpu_sc as plsc

_B = 16384
_L = 50
_D = 64
_NC = 2
_NS = 16
_NW = _NC * _NS
_U = 16
_NI = _U * _L
_UPW = _B // _NW
_NCH = _UPW // _U
_PST = 17
_PRS = 64 * _PST + 16
_GR = 80
_NG = _NI // _GR


def _sc_body(users_hbm, items_hbm, ut_hbm, it_hbm, out_hbm,
             uidx_v, iidx_v, urows_v, irows_v, prow_v, scores_v,
             isem0, isem1, rsem0, rsem1, osem0, osem1):
    wid = lax.axis_index("s") * _NC + lax.axis_index("c")
    base_row = wid * _UPW
    isems = (isem0, isem1)
    rsems = (rsem0, rsem1)
    osems = (osem0, osem1)

    def issue_idx(ch, buf):
        row0 = base_row + ch * _U
        pltpu.async_copy(users_hbm.at[pl.ds(row0, _U)], uidx_v.at[buf],
                         isems[buf])
        pltpu.async_copy(items_hbm.at[pl.ds(row0 * _L, _NI)], iidx_v.at[buf],
                         isems[buf])

    def drain_idx(buf):
        pltpu.make_async_copy(users_hbm.at[pl.ds(0, _U)], uidx_v.at[buf],
                              isems[buf]).wait()
        pltpu.make_async_copy(items_hbm.at[pl.ds(0, _NI)], iidx_v.at[buf],
                              isems[buf]).wait()

    def issue_rows(buf):
        pltpu.async_copy(ut_hbm.at[uidx_v.at[buf]], urows_v.at[buf],
                         rsems[buf])
        for g in range(_NG):
            pltpu.async_copy(
                it_hbm.at[iidx_v.at[buf, pl.ds(g * _GR, _GR)]],
                irows_v.at[buf, pl.ds(g * _GR, _GR)], rsems[buf])

    def drain_rows(buf):
        pltpu.make_async_copy(ut_hbm.at[pl.ds(0, _U)], urows_v.at[buf],
                              rsems[buf]).wait()
        pltpu.make_async_copy(it_hbm.at[pl.ds(0, _NI)], irows_v.at[buf],
                              rsems[buf]).wait()

    def drain_out(buf):
        pltpu.make_async_copy(scores_v.at[buf], out_hbm.at[pl.ds(0, _U)],
                              osems[buf]).wait()

    lane17 = lax.iota(jnp.int32, 16) * _PST

    def compute(ch, buf):
        def user_body(c, carry2):
            u0 = urows_v[buf, c, pl.ds(0, 16)]
            u1 = urows_v[buf, c, pl.ds(16, 16)]
            u2 = urows_v[buf, c, pl.ds(32, 16)]
            u3 = urows_v[buf, c, pl.ds(48, 16)]

            @plsc.parallel_loop(0, _L, unroll=5)
            def _(l):
                n = c * _L + l
                p = ((irows_v[buf, n, pl.ds(0, 16)] * u0
                      + irows_v[buf, n, pl.ds(16, 16)] * u1)
                     + (irows_v[buf, n, pl.ds(32, 16)] * u2
                        + irows_v[buf, n, pl.ds(48, 16)] * u3))
                prow_v[buf, pl.ds(l * _PST, 16)] = p

            @plsc.parallel_loop(0, 4)
            def _(g):
                col0 = g * (16 * _PST)
                acc = plsc.load_gather(prow_v.at[buf], [lane17 + col0])
                for d in range(1, 16):
                    acc = acc + plsc.load_gather(prow_v.at[buf],
                                                 [lane17 + (col0 + d)])
                scores_v[buf, c, pl.ds(g * 16, 16)] = acc
            return carry2

        lax.fori_loop(0, _U, user_body, 0)
        pltpu.async_copy(scores_v.at[buf],
                         out_hbm.at[pl.ds(base_row + ch * _U, _U)],
                         osems[buf])

    def step(ch, buf, first):
        nbuf = 1 - buf
        drain_idx(nbuf)
        issue_rows(nbuf)
        drain_rows(buf)
        issue_idx(jnp.minimum(ch + 2, _NCH - 1), buf)
        if not first:
            drain_out(buf)
        compute(ch, buf)

    issue_idx(0, 0)
    issue_idx(1, 1)
    drain_idx(0)
    issue_rows(0)

    step(0, 0, True)
    step(1, 1, True)

    def pair_body(i, carry):
        step(2 * i, 0, False)
        step(2 * i + 1, 1, False)
        return carry

    lax.fori_loop(1, _NCH // 2, pair_body, 0)

    drain_idx(1)
    drain_rows(0)
    drain_out(0)
    drain_out(1)


def kernel(users, items, user_table, item_table):
    items_flat = items.reshape(-1)
    mesh = plsc.VectorSubcoreMesh(core_axis_name="c", subcore_axis_name="s")
    out64 = pl.kernel(
        _sc_body,
        mesh=mesh,
        compiler_params=pltpu.CompilerParams(
            needs_layout_passes=False, use_tc_tiling_on_sc=False),
        out_type=jax.ShapeDtypeStruct((_B, _D), jnp.float32),
        scratch_types=[
            pltpu.VMEM((2, _U), jnp.int32),
            pltpu.VMEM((2, _NI), jnp.int32),
            pltpu.VMEM((2, _U, _D), jnp.float32),
            pltpu.VMEM((2, _NI, _D), jnp.float32),
            pltpu.VMEM((2, _PRS), jnp.float32),
            pltpu.VMEM((2, _U, _D), jnp.float32),
            pltpu.SemaphoreType.DMA,
            pltpu.SemaphoreType.DMA,
            pltpu.SemaphoreType.DMA,
            pltpu.SemaphoreType.DMA,
            pltpu.SemaphoreType.DMA,
            pltpu.SemaphoreType.DMA,
        ],
    )(users, items_flat, user_table, item_table)
    return out64[:, :_L]


# speedup vs baseline: 2.5978x; 1.0027x over previous
import jax
import jax.numpy as jnp
from jax import lax
from jax.experimental import pallas as pl
from jax.experimental.pallas import tpu as pltpu
from jax.experimental.pallas import tpu_sc as plsc

_B = 16384
_L = 50
_D = 64
_NC = 2
_NS = 16
_NW = _NC * _NS
_U = 16
_NI = _U * _L
_UPW = _B // _NW
_NCH = _UPW // _U
_PST = 17
_PRS = 64 * _PST + 16
_GR = 80
_NG = _NI // _GR
_GL = _L // _NG


def _sc_body(users_hbm, items_hbm, ut_hbm, it_hbm, out_hbm,
             uidx_v, iidx_v, iidx1_v, urows_v, irows_v, prow_v, scores_v,
             isem0, isem1, rsem0, rsem1, osem0, osem1):
    wid = lax.axis_index("s") * _NC + lax.axis_index("c")
    base_row = wid * _UPW
    isems = (isem0, isem1)
    rsems = (rsem0, rsem1)
    osems = (osem0, osem1)

    def issue_idx(ch, buf):
        row0 = base_row + ch * _U
        pltpu.async_copy(users_hbm.at[pl.ds(row0, _U)], uidx_v.at[buf],
                         isems[buf])
        pltpu.async_copy(items_hbm.at[:, pl.ds(row0, _U)], iidx_v.at[buf],
                         isems[buf])

    def drain_idx(buf):
        pltpu.make_async_copy(users_hbm.at[pl.ds(0, _U)], uidx_v.at[buf],
                              isems[buf]).wait()
        pltpu.make_async_copy(items_hbm.at[:, pl.ds(0, _U)], iidx_v.at[buf],
                              isems[buf]).wait()

    def issue_rows(buf):
        pltpu.async_copy(ut_hbm.at[uidx_v.at[buf]], urows_v.at[buf],
                         rsems[buf])
        @plsc.parallel_loop(0, _L, unroll=5)
        def _(l):
            iidx1_v[buf, pl.ds(l * _U, _U)] = iidx_v[buf, l, pl.ds(0, _U)]
        for g in range(_NG):
            pltpu.async_copy(
                it_hbm.at[iidx1_v.at[buf, pl.ds(g * _GR, _GR)]],
                irows_v.at[buf, pl.ds(g * _GR, _GR)], rsems[buf])

    def drain_rows(buf):
        pltpu.make_async_copy(ut_hbm.at[pl.ds(0, _U)], urows_v.at[buf],
                              rsems[buf]).wait()
        pltpu.make_async_copy(it_hbm.at[pl.ds(0, _NI)], irows_v.at[buf],
                              rsems[buf]).wait()

    def drain_out(buf):
        pltpu.make_async_copy(scores_v.at[buf], out_hbm.at[pl.ds(0, _U)],
                              osems[buf]).wait()

    lane17 = lax.iota(jnp.int32, 16) * _PST

    def compute(ch, buf):
        def user_body(c, carry2):
            u0 = urows_v[buf, c, pl.ds(0, 16)]
            u1 = urows_v[buf, c, pl.ds(16, 16)]
            u2 = urows_v[buf, c, pl.ds(32, 16)]
            u3 = urows_v[buf, c, pl.ds(48, 16)]

            @plsc.parallel_loop(0, _L, unroll=5)
            def _(l):
                n = l * _U + c
                p = ((irows_v[buf, n, pl.ds(0, 16)] * u0
                      + irows_v[buf, n, pl.ds(16, 16)] * u1)
                     + (irows_v[buf, n, pl.ds(32, 16)] * u2
                        + irows_v[buf, n, pl.ds(48, 16)] * u3))
                prow_v[buf, pl.ds(l * _PST, 16)] = p

            @plsc.parallel_loop(0, 4)
            def _(g):
                col0 = g * (16 * _PST)
                acc = plsc.load_gather(prow_v.at[buf], [lane17 + col0])
                for d in range(1, 16):
                    acc = acc + plsc.load_gather(prow_v.at[buf],
                                                 [lane17 + (col0 + d)])
                scores_v[buf, c, pl.ds(g * 16, 16)] = acc
            return carry2

        lax.fori_loop(0, _U, user_body, 0)
        pltpu.async_copy(scores_v.at[buf],
                         out_hbm.at[pl.ds(base_row + ch * _U, _U)],
                         osems[buf])

    def step(ch, buf, first):
        nbuf = 1 - buf
        drain_idx(nbuf)
        issue_rows(nbuf)
        drain_rows(buf)
        issue_idx(jnp.minimum(ch + 2, _NCH - 1), buf)
        if not first:
            drain_out(buf)
        compute(ch, buf)

    issue_idx(0, 0)
    issue_idx(1, 1)
    drain_idx(0)
    issue_rows(0)

    step(0, 0, True)
    step(1, 1, True)

    def pair_body(i, carry):
        step(2 * i, 0, False)
        step(2 * i + 1, 1, False)
        return carry

    lax.fori_loop(1, _NCH // 2, pair_body, 0)

    drain_idx(1)
    drain_rows(0)
    drain_out(0)
    drain_out(1)


def kernel(users, items, user_table, item_table):
    items_t = items.T
    mesh = plsc.VectorSubcoreMesh(core_axis_name="c", subcore_axis_name="s")
    out64 = pl.kernel(
        _sc_body,
        mesh=mesh,
        compiler_params=pltpu.CompilerParams(
            needs_layout_passes=False, use_tc_tiling_on_sc=False),
        out_type=jax.ShapeDtypeStruct((_B, _D), jnp.float32),
        scratch_types=[
            pltpu.VMEM((2, _U), jnp.int32),
            pltpu.VMEM((2, _L, _U), jnp.int32),
            pltpu.VMEM((2, _NI), jnp.int32),
            pltpu.VMEM((2, _U, _D), jnp.float32),
            pltpu.VMEM((2, _NI, _D), jnp.float32),
            pltpu.VMEM((2, _PRS), jnp.float32),
            pltpu.VMEM((2, _U, _D), jnp.float32),
            pltpu.SemaphoreType.DMA,
            pltpu.SemaphoreType.DMA,
            pltpu.SemaphoreType.DMA,
            pltpu.SemaphoreType.DMA,
            pltpu.SemaphoreType.DMA,
            pltpu.SemaphoreType.DMA,
        ],
    )(users, items_t, user_table, item_table)
    return out64[:, :_L]


# speedup vs baseline: 2.7501x; 1.0586x over previous
import jax
import jax.numpy as jnp
from jax import lax
from jax.experimental import pallas as pl
from jax.experimental.pallas import tpu as pltpu
from jax.experimental.pallas import tpu_sc as plsc

_B = 16384
_L = 50
_D = 64
_NC = 2
_NS = 16
_NW = _NC * _NS
_U = 16
_NI = _U * _L
_UPW = _B // _NW
_NCH = _UPW // _U
_PST = 17
_PRS = 64 * _PST + 16
_GR = 80
_NG = _NI // _GR
_GL = _L // _NG


def _sc_body(users_hbm, items_hbm, ut_hbm, it_hbm, out_hbm,
             uidx_v, iidx_v, iidx1_v, urows_v, irows_v, prow_v, scores_v,
             isem0, isem1, rsem0, rsem1, osem0, osem1):
    wid = lax.axis_index("s") * _NC + lax.axis_index("c")
    base_row = wid * _UPW
    isems = (isem0, isem1)
    rsems = (rsem0, rsem1)
    osems = (osem0, osem1)

    def issue_idx(ch, buf):
        row0 = base_row + ch * _U
        pltpu.async_copy(users_hbm.at[pl.ds(row0, _U)], uidx_v.at[buf],
                         isems[buf])
        pltpu.async_copy(items_hbm.at[:, pl.ds(row0, _U)], iidx_v.at[buf],
                         isems[buf])

    def drain_idx(buf):
        pltpu.make_async_copy(users_hbm.at[pl.ds(0, _U)], uidx_v.at[buf],
                              isems[buf]).wait()
        pltpu.make_async_copy(items_hbm.at[:, pl.ds(0, _U)], iidx_v.at[buf],
                              isems[buf]).wait()

    def issue_rows(buf):
        uidx_v[buf, pl.ds(0, _U)] = uidx_v[buf, pl.ds(0, _U)] * 2
        pltpu.async_copy(ut_hbm.at[uidx_v.at[buf]], urows_v.at[buf],
                         rsems[buf])
        @plsc.parallel_loop(0, _L, unroll=5)
        def _(l):
            iidx1_v[buf, pl.ds(l * _U, _U)] = iidx_v[buf, l, pl.ds(0, _U)] * 2
        for g in range(_NG):
            pltpu.async_copy(
                it_hbm.at[iidx1_v.at[buf, pl.ds(g * _GR, _GR)]],
                irows_v.at[buf, pl.ds(g * _GR, _GR)], rsems[buf])

    def drain_rows(buf):
        pltpu.make_async_copy(ut_hbm.at[pl.ds(0, _U)], urows_v.at[buf],
                              rsems[buf]).wait()
        pltpu.make_async_copy(it_hbm.at[pl.ds(0, _NI)], irows_v.at[buf],
                              rsems[buf]).wait()

    def drain_out(buf):
        pltpu.make_async_copy(scores_v.at[buf], out_hbm.at[pl.ds(0, _U)],
                              osems[buf]).wait()

    lane17 = lax.iota(jnp.int32, 16) * _PST

    def compute(ch, buf):
        def user_body(c, carry2):
            u0 = urows_v[buf, c, pl.ds(0, 16)]
            u1 = urows_v[buf, c, pl.ds(16, 16)]
            u2 = urows_v[buf, c, pl.ds(32, 16)]
            u3 = urows_v[buf, c, pl.ds(48, 16)]

            @plsc.parallel_loop(0, _L, unroll=5)
            def _(l):
                n = l * _U + c
                p = ((irows_v[buf, n, pl.ds(0, 16)] * u0
                      + irows_v[buf, n, pl.ds(16, 16)] * u1)
                     + (irows_v[buf, n, pl.ds(32, 16)] * u2
                        + irows_v[buf, n, pl.ds(48, 16)] * u3))
                prow_v[buf, pl.ds(l * _PST, 16)] = p

            @plsc.parallel_loop(0, 4)
            def _(g):
                col0 = g * (16 * _PST)
                acc = plsc.load_gather(prow_v.at[buf], [lane17 + col0])
                for d in range(1, 16):
                    acc = acc + plsc.load_gather(prow_v.at[buf],
                                                 [lane17 + (col0 + d)])
                scores_v[buf, c, pl.ds(g * 16, 16)] = acc
            return carry2

        lax.fori_loop(0, _U, user_body, 0)
        pltpu.async_copy(scores_v.at[buf],
                         out_hbm.at[pl.ds(base_row + ch * _U, _U)],
                         osems[buf])

    def step(ch, buf, first):
        nbuf = 1 - buf
        drain_idx(nbuf)
        issue_rows(nbuf)
        drain_rows(buf)
        issue_idx(jnp.minimum(ch + 2, _NCH - 1), buf)
        if not first:
            drain_out(buf)
        compute(ch, buf)

    issue_idx(0, 0)
    issue_idx(1, 1)
    drain_idx(0)
    issue_rows(0)

    step(0, 0, True)
    step(1, 1, True)

    def pair_body(i, carry):
        step(2 * i, 0, False)
        step(2 * i + 1, 1, False)
        return carry

    lax.fori_loop(1, _NCH // 2, pair_body, 0)

    drain_idx(1)
    drain_rows(0)
    drain_out(0)
    drain_out(1)


def kernel(users, items, user_table, item_table):
    items_t = items.T
    ut_p = jnp.pad(user_table, ((0, 0), (0, _D))).reshape(-1, _D)
    it_p = jnp.pad(item_table, ((0, 0), (0, _D))).reshape(-1, _D)
    mesh = plsc.VectorSubcoreMesh(core_axis_name="c", subcore_axis_name="s")
    out64 = pl.kernel(
        _sc_body,
        mesh=mesh,
        compiler_params=pltpu.CompilerParams(
            needs_layout_passes=False, use_tc_tiling_on_sc=False),
        out_type=jax.ShapeDtypeStruct((_B, _D), jnp.float32),
        scratch_types=[
            pltpu.VMEM((2, _U), jnp.int32),
            pltpu.VMEM((2, _L, _U), jnp.int32),
            pltpu.VMEM((2, _NI), jnp.int32),
            pltpu.VMEM((2, _U, _D), jnp.float32),
            pltpu.VMEM((2, _NI, _D), jnp.float32),
            pltpu.VMEM((2, _PRS), jnp.float32),
            pltpu.VMEM((2, _U, _D), jnp.float32),
            pltpu.SemaphoreType.DMA,
            pltpu.SemaphoreType.DMA,
            pltpu.SemaphoreType.DMA,
            pltpu.SemaphoreType.DMA,
            pltpu.SemaphoreType.DMA,
            pltpu.SemaphoreType.DMA,
        ],
    )(users, items_t, ut_p, it_p)
    return out64[:, :_L]
